# baseline (speedup 1.0000x reference)
import functools

import jax
import jax.numpy as jnp
from jax import lax
from jax.experimental import pallas as pl
from jax.experimental.pallas import tpu as pltpu
from jax.experimental.pallas import tpu_sc as plsc

B = 1024
K = 200
V = 100000

_NUM_CORES = 2
_NUM_SUBCORES = 16
NW = _NUM_CORES * _NUM_SUBCORES
PER_W = (B // NW) * K
ROWS_PER_W = B // NW
LANES = 16
VREGS_PER_ROW = -(-K // LANES)
CHUNK = 128
N_CHUNKS = PER_W // CHUNK
FIRE = 10
PAD = 16


@functools.partial(
    pl.kernel,
    out_type=jax.ShapeDtypeStruct((B * K,), jnp.float32),
    mesh=plsc.VectorSubcoreMesh(core_axis_name="c", subcore_axis_name="s"),
    scratch_types=[
        pltpu.VMEM((PER_W + PAD,), jnp.int32),
        pltpu.VMEM((PER_W + PAD,), jnp.int32),
        pltpu.VMEM((PER_W,), jnp.float32),
        pltpu.SemaphoreType.DMA,
    ],
)
def _sc_gather(scores_hbm, cids_hbm, out_hbm, cid_v, flat_v, out_v, sem):
    wid = lax.axis_index("s") * _NUM_CORES + lax.axis_index("c")
    base = pl.multiple_of(wid * PER_W, PER_W)
    pltpu.sync_copy(cids_hbm.at[pl.ds(base, PER_W)], cid_v.at[pl.ds(0, PER_W)])

    def idx_body(r, carry):
        row_off = (wid * ROWS_PER_W + r) * V
        o_row = pl.multiple_of(r * K, 8)
        for j in range(VREGS_PER_ROW):
            sl = pl.ds(o_row + j * LANES, LANES)
            flat_v[sl] = cid_v[sl] + row_off
        return carry

    lax.fori_loop(0, ROWS_PER_W, idx_body, 0)

    def gather_body(g, carry):
        copies = []
        for j in range(FIRE):
            o = pl.multiple_of((g * FIRE + j) * CHUNK, CHUNK)
            copies.append(pltpu.make_async_copy(
                scores_hbm.at[flat_v.at[pl.ds(o, CHUNK)]],
                out_v.at[pl.ds(o, CHUNK)],
                sem,
            ))
        for c in copies:
            c.start()
        for c in copies:
            c.wait()
        return carry

    lax.fori_loop(0, N_CHUNKS // FIRE, gather_body, 0)
    pltpu.sync_copy(out_v, out_hbm.at[pl.ds(base, PER_W)])


def kernel(scores, candidate_ids):
    out = _sc_gather(
        jnp.reshape(scores, (B * V,)),
        jnp.reshape(candidate_ids, (B * K,)),
    )
    return jnp.reshape(out, (B, K))


# speedup vs baseline: 27.4906x; 27.4906x over previous
import functools

import jax
import jax.numpy as jnp
from jax import lax
from jax.experimental import pallas as pl
from jax.experimental.pallas import tpu as pltpu
from jax.experimental.pallas import tpu_sc as plsc

B = 1024
K = 200
V = 100000
N = B * K

_NUM_CORES = 2
_NUM_SUBCORES = 16
NW = _NUM_CORES * _NUM_SUBCORES
PER_W = N // NW
LANES = 16
VREGS_PER_W = PER_W // LANES
UNROLL = 16
CHUNK = 128
N_CHUNKS = PER_W // CHUNK
FIRE = 10


@functools.partial(
    pl.kernel,
    out_type=jax.ShapeDtypeStruct((N,), jnp.float32),
    mesh=plsc.VectorSubcoreMesh(core_axis_name="c", subcore_axis_name="s"),
    scratch_types=[
        pltpu.VMEM((PER_W,), jnp.int32),
        pltpu.VMEM((PER_W,), jnp.float32),
        pltpu.SemaphoreType.DMA,
    ],
)
def _sc_gather(scores_hbm, cids_hbm, out_hbm, idx_v, out_v, sem):
    wid = lax.axis_index("s") * _NUM_CORES + lax.axis_index("c")
    base = pl.multiple_of(wid * PER_W, PER_W)
    pltpu.sync_copy(cids_hbm.at[pl.ds(base, PER_W)], idx_v)

    lane = lax.iota(jnp.int32, LANES)

    def addr_body(g, carry):
        for j in range(UNROLL):
            t = g * UNROLL + j
            sl = pl.ds(t * LANES, LANES)
            p0 = base + t * LANES
            v = idx_v[sl]
            idx_v[sl] = (
                ((v >> 3) << 13)
                + ((v & 7) << 7)
                + ((p0 & 0x1C00) + (p0 & 127) + lane)
            )
        return carry

    lax.fori_loop(0, VREGS_PER_W // UNROLL, addr_body, 0)

    def gather_body(g, carry):
        copies = []
        for j in range(FIRE):
            o = pl.multiple_of((g * FIRE + j) * CHUNK, CHUNK)
            copies.append(pltpu.make_async_copy(
                scores_hbm.at[idx_v.at[pl.ds(o, CHUNK)]],
                out_v.at[pl.ds(o, CHUNK)],
                sem,
            ))
        for c in copies:
            c.start()
        for c in copies:
            c.wait()
        return carry

    lax.fori_loop(0, N_CHUNKS // FIRE, gather_body, 0)
    pltpu.sync_copy(out_v, out_hbm.at[pl.ds(base, PER_W)])


def kernel(scores, candidate_ids):
    s_flat = (
        jnp.squeeze(scores, axis=1).T
        .reshape(V // 8, 8, B // 128, 128)
        .transpose(0, 2, 1, 3)
        .reshape(-1)
    )
    c_flat = (
        candidate_ids.T
        .reshape(K // 8, 8, B // 128, 128)
        .transpose(0, 2, 1, 3)
        .reshape(-1)
    )
    out_flat = _sc_gather(s_flat, c_flat)
    return (
        out_flat
        .reshape(K // 8, B // 128, 8, 128)
        .transpose(0, 2, 1, 3)
        .reshape(K, B)
        .T
    )
